# baseline (speedup 1.0000x reference)
import functools

import jax
import jax.numpy as jnp
from jax import lax
from jax.experimental import pallas as pl
from jax.experimental.pallas import tpu as pltpu
from jax.experimental.pallas import tpu_sc as plsc

N = 10000
E = 320000
H = 16
BATCH = 4096
F32 = jnp.float32

NC, NS = 2, 16
NW = NC * NS
EW = E // NW
KB = 80
NBLK = EW // KB
NPAD = 10240
NPS = NPAD // NS
BW = BATCH // NW

_MLP_BE = 2560
_NODE_BN = 1000

_LP = (3.5075520531946403e-06, 0.9997924357285933, -0.49697791116741225,
       0.31459053536992065, -0.18878267361890674, 0.08172680837331736,
       -0.017208061120537015)


def _mlp_body(ea, We1, be1, We2, be2, Wt, bt, t):
    bf = jnp.bfloat16
    h1 = jnp.maximum(jnp.dot(ea[...].astype(bf), We1[...].astype(bf),
                             preferred_element_type=F32) + be1[...], 0.0)
    h2 = jnp.maximum(jnp.dot(h1.astype(bf), We2[...].astype(bf),
                             preferred_element_type=F32) + be2[...], 0.0)
    t[...] = jnp.dot(h2.astype(bf), Wt[...].astype(bf),
                     preferred_element_type=F32) + bt[...]


def _run_mlp(ea2, We1, be1, We2, be2, Wt, bt):
    be = _MLP_BE // 2
    E2 = E // 2
    full = lambda shape: pl.BlockSpec(shape, lambda e: (0, 0))
    return pl.pallas_call(
        _mlp_body,
        grid=(E2 // be,),
        in_specs=[
            pl.BlockSpec((be, 32), lambda e: (e, 0)),
            full((32, 512)), full((1, 512)),
            full((512, 256)), full((1, 256)),
            full((256, 128)), full((1, 128)),
        ],
        out_specs=pl.BlockSpec((be, 128), lambda e: (e, 0)),
        out_shape=jax.ShapeDtypeStruct((E2, 128), F32),
    )(ea2, We1, be1, We2, be2, Wt, bt)


def _b1_body(xur, Wur, bur, Aur, xir, Wir, bir, Air, xu_o, xi_o, U1_o, I1_o):
    xu = jnp.dot(xur[...], Wur[...], preferred_element_type=F32) + bur[...]
    xi = jnp.dot(xir[...], Wir[...], preferred_element_type=F32) + bir[...]
    xu_o[...] = xu
    xi_o[...] = xi
    U1_o[...] = jnp.dot(xu, Aur[...], preferred_element_type=F32)
    I1_o[...] = jnp.dot(xi, Air[...], preferred_element_type=F32)


def _run_b1(x_user, Wu, bu, Au, x_item, Wi, bi, Ai):
    bn = _NODE_BN
    full = lambda shape: pl.BlockSpec(shape, lambda r: (0, 0))
    row = lambda w: pl.BlockSpec((bn, w), lambda r: (r, 0))
    return pl.pallas_call(
        _b1_body,
        grid=(N // bn,),
        in_specs=[row(128), full((128, 16)), full((1, 16)), full((16, 64)),
                  row(128), full((128, 16)), full((1, 16)), full((16, 64))],
        out_specs=[row(16), row(16), row(64), row(64)],
        out_shape=[jax.ShapeDtypeStruct((N, 16), F32),
                   jax.ShapeDtypeStruct((N, 16), F32),
                   jax.ShapeDtypeStruct((N, 64), F32),
                   jax.ShapeDtypeStruct((N, 64), F32)],
    )(x_user, Wu, bu, Au, x_item, Wi, bi, Ai)


def _b2_body(accU, accI, xu, xi, gU, bU, gI, bI, Au2, Ai2,
             xu1_o, xi1_o, U2_o, I2_o, cu_o, ci_o):
    sU = accU[0] + accU[1]
    sI = accI[0] + accI[1]
    cntU = jnp.maximum(sU[:, 16:17], 1.0)
    cntI = jnp.maximum(sI[:, 16:17], 1.0)
    xu1 = sU[:, :16] / cntU * gU[...] + bU[...] + xu[...]
    xi1 = sI[:, :16] / cntI * gI[...] + bI[...] + xi[...]
    xu1_o[...] = xu1
    xi1_o[...] = xi1
    U2_o[...] = jnp.dot(xu1, Au2[...], preferred_element_type=F32)
    I2_o[...] = jnp.dot(xi1, Ai2[...], preferred_element_type=F32)
    cu_o[...] = cntU
    ci_o[...] = cntI


def _run_b2(accU, accI, xu, xi, gU, bU, gI, bI, Au2, Ai2):
    bn = _NODE_BN
    full = lambda shape: pl.BlockSpec(shape, lambda r: (0, 0))
    row = lambda w: pl.BlockSpec((bn, w), lambda r: (r, 0))
    acc = pl.BlockSpec((2, bn, 32), lambda r: (0, r, 0))
    return pl.pallas_call(
        _b2_body,
        grid=(N // bn,),
        in_specs=[acc, acc, row(16), row(16), full((1, 16)), full((1, 16)),
                  full((1, 16)), full((1, 16)), full((16, 64)), full((16, 64))],
        out_specs=[row(16), row(16), row(64), row(64), row(1), row(1)],
        out_shape=[jax.ShapeDtypeStruct((N, 16), F32),
                   jax.ShapeDtypeStruct((N, 16), F32),
                   jax.ShapeDtypeStruct((N, 64), F32),
                   jax.ShapeDtypeStruct((N, 64), F32),
                   jax.ShapeDtypeStruct((N, 1), F32),
                   jax.ShapeDtypeStruct((N, 1), F32)],
    )(accU, accI, xu, xi, gU, bU, gI, bI, Au2, Ai2)


def _b3_body(acc2U, acc2I, cu, ci, xu1, xi1, gU, bU, gI, bI,
             wu, wi, bens, pu_o, pi_o):
    xu2 = (acc2U[0] + acc2U[1]) / cu[...] * gU[...] + bU[...] + xu1[...]
    xi2 = (acc2I[0] + acc2I[1]) / ci[...] * gI[...] + bI[...] + xi1[...]
    pu_o[...] = jnp.dot(xu2, wu[...], preferred_element_type=F32) + bens[...]
    pi_o[...] = jnp.dot(xi2, wi[...], preferred_element_type=F32)


def _run_b3(acc2U, acc2I, cu, ci, xu1, xi1, gU, bU, gI, bI, wu, wi, bens):
    bn = _NODE_BN
    full = lambda shape: pl.BlockSpec(shape, lambda r: (0, 0))
    row = lambda w: pl.BlockSpec((bn, w), lambda r: (r, 0))
    acc = pl.BlockSpec((2, bn, 16), lambda r: (0, r, 0))
    return pl.pallas_call(
        _b3_body,
        grid=(N // bn,),
        in_specs=[acc, acc, row(1), row(1), row(16), row(16),
                  full((1, 16)), full((1, 16)), full((1, 16)), full((1, 16)),
                  full((16, 1)), full((16, 1)), full((1, 1))],
        out_specs=[row(1), row(1)],
        out_shape=[jax.ShapeDtypeStruct((N, 1), F32),
                   jax.ShapeDtypeStruct((N, 1), F32)],
    )(acc2U, acc2I, cu, ci, xu1, xi1, gU, bU, gI, bI, wu, wi, bens)


def _sigmoid(f):
    return 1.0 / (1.0 + jnp.exp(-f))


def _softplus(s):
    u = jnp.exp(-jnp.abs(s))
    p = _LP[6]
    for c in (_LP[5], _LP[4], _LP[3], _LP[2], _LP[1], _LP[0]):
        p = p * u + c
    return jnp.maximum(s, 0.0) + p


def _msg(f, s):
    return _sigmoid(f) * _softplus(s)


def _make_conv_kernel(width):
    mesh = plsc.VectorSubcoreMesh(core_axis_name="c", subcore_axis_name="s",
                                  num_cores=NC, num_subcores=NS)

    KB2 = KB // 2
    EW2 = EW // 2

    @functools.partial(
        pl.kernel,
        mesh=mesh,
        compiler_params=pltpu.CompilerParams(use_tc_tiling_on_sc=False),
        out_type=[jax.ShapeDtypeStruct((NC, NPAD, width), F32),
                  jax.ShapeDtypeStruct((NC, NPAD, width), F32)],
        scratch_types=[
            pltpu.VMEM((NBLK, KB), jnp.int32),
            pltpu.VMEM((NBLK, KB), jnp.int32),
            pltpu.VMEM((KB, 64), F32), pltpu.VMEM((KB, 64), F32),
            pltpu.VMEM((KB, 64), F32), pltpu.VMEM((KB, 64), F32),
            pltpu.VMEM((KB2, 128), F32), pltpu.VMEM((KB2, 128), F32),
            pltpu.VMEM((KB, width), F32), pltpu.VMEM((KB, width), F32),
            pltpu.VMEM((KB, width), F32), pltpu.VMEM((KB, width), F32),
            pltpu.VMEM((NPS, width), F32),
            pltpu.VMEM_SHARED((NPAD, width), F32),
            pltpu.VMEM_SHARED((NPAD, width), F32),
            pltpu.SemaphoreType.DMA, pltpu.SemaphoreType.DMA,
            pltpu.SemaphoreType.DMA, pltpu.SemaphoreType.DMA,
            pltpu.SemaphoreType.DMA, pltpu.SemaphoreType.DMA,
            pltpu.SemaphoreType.DMA, pltpu.SemaphoreType.DMA,
            pltpu.SemaphoreType.DMA, pltpu.SemaphoreType.DMA,
        ],
    )
    def conv(uidx_hbm, iidx_hbm, t_hbm, U_hbm, I_hbm,
             accU_hbm, accI_hbm,
             idxU, idxI, rU0, rU1, rI0, rI1, tb0, tb1,
             mU0, mU1, mI0, mI1, zbuf,
             sAccU, sAccI, sU0, sU1, sI0, sI1, sT0, sT1,
             sSU0, sSU1, sSI0, sSI1):
        cid = lax.axis_index("c")
        sid = lax.axis_index("s")
        wid = sid * NC + cid

        zero16 = jnp.zeros((16,), F32)

        @plsc.parallel_loop(0, NPS, unroll=8)
        def zrow(r):
            for j in range(width // 16):
                zbuf[r, pl.ds(j * 16, 16)] = zero16

        zb = sid * NPS
        pltpu.sync_copy(zbuf, sAccU.at[pl.ds(zb, NPS)])
        pltpu.sync_copy(zbuf, sAccI.at[pl.ds(zb, NPS)])

        pltpu.sync_copy(uidx_hbm.at[wid], idxU)
        pltpu.sync_copy(iidx_hbm.at[wid], idxI)

        if width == 32:
            lanes = lax.iota(jnp.int32, 16)
            onehot = jnp.where(lanes == 0, 1.0, 0.0).astype(F32)

            @plsc.parallel_loop(0, KB, unroll=8)
            def initrow(k):
                for m in (mU0, mU1, mI0, mI1):
                    m[k, pl.ds(16, 16)] = onehot

        plsc.subcore_barrier()

        pbase = wid * EW2

        def gathers(b, rU, rI, tb, sU, sI, sT):
            gu = pltpu.make_async_copy(U_hbm.at[idxU.at[b]], rU, sU)
            gi = pltpu.make_async_copy(I_hbm.at[idxI.at[b]], rI, sI)
            gt = pltpu.make_async_copy(t_hbm.at[pl.ds(pbase + b * KB2, KB2)],
                                       tb, sT)
            return gu, gi, gt

        def issue(b, rU, rI, tb, sU, sI, sT):
            for c in gathers(b, rU, rI, tb, sU, sI, sT):
                c.start()

        def wait(b, rU, rI, tb, sU, sI, sT):
            for c in gathers(b, rU, rI, tb, sU, sI, sT):
                c.wait()

        def scat(b, mU, mI, sSU, sSI):
            cu = pltpu.make_async_copy(mU, sAccU.at[idxU.at[b]], sSU)
            ci = pltpu.make_async_copy(mI, sAccI.at[idxI.at[b]], sSI)
            return cu, ci

        def compute_scatter(b, rU, rI, tb, mU, mI, sSU, sSI):
            msgU, msgI = mU, mI

            @plsc.parallel_loop(0, KB2, unroll=8)
            def pair(p):
                for h in range(2):
                    k = p * 2 + h
                    tfu = tb[p, pl.ds(64 * h + 0, 16)]
                    tsu = tb[p, pl.ds(64 * h + 16, 16)]
                    tfi = tb[p, pl.ds(64 * h + 32, 16)]
                    tsi = tb[p, pl.ds(64 * h + 48, 16)]
                    ufd = rU[k, pl.ds(0, 16)]
                    usd = rU[k, pl.ds(16, 16)]
                    ufs = rU[k, pl.ds(32, 16)]
                    uss = rU[k, pl.ds(48, 16)]
                    ifd = rI[k, pl.ds(0, 16)]
                    isd = rI[k, pl.ds(16, 16)]
                    ifs = rI[k, pl.ds(32, 16)]
                    iss = rI[k, pl.ds(48, 16)]
                    msgI[k, pl.ds(0, 16)] = _msg(tfu + ifd + ufs,
                                                 tsu + isd + uss)
                    msgU[k, pl.ds(0, 16)] = _msg(tfi + ufd + ifs,
                                                 tsi + usd + iss)
            pltpu.async_copy(msgU, sAccU.at[idxU.at[b]], sSU, add=True)
            pltpu.async_copy(msgI, sAccI.at[idxI.at[b]], sSI, add=True)

        issue(0, rU0, rI0, tb0, sU0, sI0, sT0)

        def pipeline(g, c):
            b0 = 2 * g
            b1 = b0 + 1
            b2 = b0 + 2
            wait(b0, rU0, rI0, tb0, sU0, sI0, sT0)
            issue(b1, rU1, rI1, tb1, sU1, sI1, sT1)

            @pl.when(g > 0)
            def _():
                for c_ in scat(b0 - 2, mU0, mI0, sSU0, sSI0):
                    c_.wait()
            compute_scatter(b0, rU0, rI0, tb0, mU0, mI0, sSU0, sSI0)
            wait(b1, rU1, rI1, tb1, sU1, sI1, sT1)
            issue(b2, rU0, rI0, tb0, sU0, sI0, sT0)

            @pl.when(g > 0)
            def _():
                for c_ in scat(b1 - 2, mU1, mI1, sSU1, sSI1):
                    c_.wait()
            compute_scatter(b1, rU1, rI1, tb1, mU1, mI1, sSU1, sSI1)
            return c
        lax.fori_loop(0, NBLK // 2, pipeline, 0)
        wait(NBLK - 1, rU0, rI0, tb0, sU0, sI0, sT0)
        for c_ in scat(NBLK - 3, mU0, mI0, sSU0, sSI0):
            c_.wait()
        compute_scatter(NBLK - 1, rU0, rI0, tb0, mU0, mI0, sSU0, sSI0)
        for c_ in scat(NBLK - 2, mU1, mI1, sSU1, sSI1):
            c_.wait()
        for c_ in scat(NBLK - 1, mU0, mI0, sSU0, sSI0):
            c_.wait()

        plsc.subcore_barrier()

        pltpu.sync_copy(sAccU.at[pl.ds(zb, NPS)],
                        accU_hbm.at[cid, pl.ds(zb, NPS)])
        pltpu.sync_copy(sAccI.at[pl.ds(zb, NPS)],
                        accI_hbm.at[cid, pl.ds(zb, NPS)])

    return conv


_make_conv_kernel = functools.lru_cache(maxsize=None)(_make_conv_kernel)


def _make_ens_kernel():
    mesh = plsc.VectorSubcoreMesh(core_axis_name="c", subcore_axis_name="s",
                                  num_cores=NC, num_subcores=NS)

    @functools.partial(
        pl.kernel,
        mesh=mesh,
        compiler_params=pltpu.CompilerParams(needs_layout_passes=False),
        out_type=jax.ShapeDtypeStruct((BATCH,), F32),
        scratch_types=[
            pltpu.VMEM((N,), F32), pltpu.VMEM((N,), F32),
            pltpu.VMEM((BW,), jnp.int32), pltpu.VMEM((BW,), jnp.int32),
            pltpu.VMEM((BW,), F32),
        ],
    )
    def ens(pu_hbm, pi_hbm, ebu_hbm, ebi_hbm, out_hbm,
            puv, piv, idxu, idxi, outv):
        cid = lax.axis_index("c")
        sid = lax.axis_index("s")
        wid = sid * NC + cid
        base = wid * BW
        pltpu.sync_copy(pu_hbm, puv)
        pltpu.sync_copy(pi_hbm, piv)
        pltpu.sync_copy(ebu_hbm.at[pl.ds(base, BW)], idxu)
        pltpu.sync_copy(ebi_hbm.at[pl.ds(base, BW)], idxi)

        def chunk(j, c):
            iu = idxu[pl.ds(j * 16, 16)]
            ii = idxi[pl.ds(j * 16, 16)]
            a = plsc.load_gather(puv, [iu])
            b = plsc.load_gather(piv, [ii])
            outv[pl.ds(j * 16, 16)] = _sigmoid(a + b)
            return c
        lax.fori_loop(0, BW // 16, chunk, 0)
        pltpu.sync_copy(outv, out_hbm.at[pl.ds(base, BW)])

    return ens


_make_ens_kernel = functools.lru_cache(maxsize=None)(_make_ens_kernel)


def kernel(x_user, x_item, edge_attr, edge_index, edge_index_batch,
           We1, be1, We2, be2, We3, be3, Wu, bu, Wi, bi,
           c1ui_Wf, c1ui_bf, c1ui_Ws, c1ui_bs, c1ui_g, c1ui_b,
           c1iu_Wf, c1iu_bf, c1iu_Ws, c1iu_bs, c1iu_g, c1iu_b,
           c2ui_Wf, c2ui_bf, c2ui_Ws, c2ui_bs, c2ui_g, c2ui_b,
           c2iu_Wf, c2iu_bf, c2iu_Ws, c2iu_bs, c2iu_g, c2iu_b,
           Wens, bens):
    def epart(Wf, Ws):
        return jnp.concatenate([Wf[2 * H:], Ws[2 * H:]], axis=1)

    S1 = jnp.concatenate([epart(c1ui_Wf, c1ui_Ws),
                          epart(c1iu_Wf, c1iu_Ws)], axis=1)
    S2 = jnp.concatenate([epart(c2ui_Wf, c2ui_Ws),
                          epart(c2iu_Wf, c2iu_Ws)], axis=1)
    Wt1 = We3 @ S1
    Wt2 = We3 @ S2
    bt1 = (be3 @ S1 + jnp.concatenate([c1ui_bf, c1ui_bs, c1iu_bf, c1iu_bs])
           )[None, :]
    bt2 = (be3 @ S2 + jnp.concatenate([c2ui_bf, c2ui_bs, c2iu_bf, c2iu_bs])
           )[None, :]

    def tables(Wf_ud, Ws_ud, Wf_du, Ws_du):
        Au = jnp.concatenate([Wf_du[:H], Ws_du[:H],
                              Wf_ud[H:2 * H], Ws_ud[H:2 * H]], axis=1)
        Ai = jnp.concatenate([Wf_ud[:H], Ws_ud[:H],
                              Wf_du[H:2 * H], Ws_du[H:2 * H]], axis=1)
        return Au, Ai

    Au1, Ai1 = tables(c1ui_Wf, c1ui_Ws, c1iu_Wf, c1iu_Ws)
    Au2, Ai2 = tables(c2ui_Wf, c2ui_Ws, c2iu_Wf, c2iu_Ws)

    inv = 1.0 / jnp.sqrt(jnp.float32(1.0 + 1e-5))
    g1U, b1U = (c1iu_g * inv)[None, :], c1iu_b[None, :]
    g1I, b1I = (c1ui_g * inv)[None, :], c1ui_b[None, :]
    g2U, b2U = (c2iu_g * inv)[None, :], c2iu_b[None, :]
    g2I, b2I = (c2ui_g * inv)[None, :], c2ui_b[None, :]

    uidx = edge_index[0].astype(jnp.int32).reshape(NW, NBLK, KB)
    iidx = edge_index[1].astype(jnp.int32).reshape(NW, NBLK, KB)
    ebu = edge_index_batch[:, 0].astype(jnp.int32)
    ebi = edge_index_batch[:, 1].astype(jnp.int32)

    eye2 = jnp.eye(2, dtype=F32)
    We1p = jnp.kron(eye2, We1)
    We2p = jnp.kron(eye2, We2)
    Wt1p = jnp.kron(eye2, Wt1)
    Wt2p = jnp.kron(eye2, Wt2)
    be1p = jnp.tile(be1, 2)[None, :]
    be2p = jnp.tile(be2, 2)[None, :]
    bt1p = jnp.tile(bt1[0], 2)[None, :]
    bt2p = jnp.tile(bt2[0], 2)[None, :]
    ea2 = edge_attr.reshape(E // 2, 32)

    t1 = _run_mlp(ea2, We1p, be1p, We2p, be2p, Wt1p, bt1p)
    xu, xi, U1, I1 = _run_b1(x_user, Wu, bu[None, :], Au1,
                             x_item, Wi, bi[None, :], Ai1)

    accU, accI = _make_conv_kernel(32)(uidx, iidx, t1, U1, I1)

    t2 = _run_mlp(ea2, We1p, be1p, We2p, be2p, Wt2p, bt2p)

    xu1, xi1, U2, I2, cu, ci = _run_b2(accU[:, :N], accI[:, :N], xu, xi,
                                       g1U, b1U, g1I, b1I, Au2, Ai2)

    acc2U, acc2I = _make_conv_kernel(16)(uidx, iidx, t2, U2, I2)

    pu, pi = _run_b3(acc2U[:, :N], acc2I[:, :N], cu, ci, xu1, xi1,
                     g2U, b2U, g2I, b2I,
                     Wens[:H], Wens[H:], bens[None, :])

    out = _make_ens_kernel()(pu.reshape(N), pi.reshape(N), ebu, ebi)
    return out.reshape(BATCH, 1)


# speedup vs baseline: 1.4997x; 1.4997x over previous
import functools

import jax
import jax.numpy as jnp
from jax import lax
from jax.experimental import pallas as pl
from jax.experimental.pallas import tpu as pltpu
from jax.experimental.pallas import tpu_sc as plsc

N = 10000
E = 320000
H = 16
BATCH = 4096
F32 = jnp.float32

NC, NS = 2, 16
NW = NC * NS
EW = E // NW
KB = 80
NBLK = EW // KB
NPAD = 10240
NPS = NPAD // NS
BW = BATCH // NW

_MLP_BE = 2560
_NODE_BN = 1000

_LP = (3.5075520531946403e-06, 0.9997924357285933, -0.49697791116741225,
       0.31459053536992065, -0.18878267361890674, 0.08172680837331736,
       -0.017208061120537015)


def _mlp_body(ea, We1, be1, We2, be2, Wt, bt, t):
    bf = jnp.bfloat16
    h1 = jnp.maximum(jnp.dot(ea[...].astype(bf), We1[...].astype(bf),
                             preferred_element_type=F32) + be1[...], 0.0)
    h2 = jnp.maximum(jnp.dot(h1.astype(bf), We2[...].astype(bf),
                             preferred_element_type=F32) + be2[...], 0.0)
    t[...] = jnp.dot(h2.astype(bf), Wt[...].astype(bf),
                     preferred_element_type=F32) + bt[...]


def _run_mlp(ea2, We1, be1, We2, be2, Wt, bt):
    be = _MLP_BE // 2
    E2 = E // 2
    full = lambda shape: pl.BlockSpec(shape, lambda e: (0, 0))
    return pl.pallas_call(
        _mlp_body,
        grid=(E2 // be,),
        compiler_params=pltpu.CompilerParams(
            allow_input_fusion=[True] + [False] * 6),
        in_specs=[
            pl.BlockSpec((be, 32), lambda e: (e, 0)),
            full((32, 512)), full((1, 512)),
            full((512, 256)), full((1, 256)),
            full((256, 128)), full((1, 128)),
        ],
        out_specs=pl.BlockSpec((be, 128), lambda e: (e, 0)),
        out_shape=jax.ShapeDtypeStruct((E2, 128), F32),
    )(ea2, We1, be1, We2, be2, Wt, bt)


def _b1_body(xur, Wur, bur, Aur, xir, Wir, bir, Air, xu_o, xi_o, U1_o, I1_o):
    xu = jnp.dot(xur[...], Wur[...], preferred_element_type=F32) + bur[...]
    xi = jnp.dot(xir[...], Wir[...], preferred_element_type=F32) + bir[...]
    xu_o[...] = xu
    xi_o[...] = xi
    U1_o[...] = jnp.dot(xu, Aur[...], preferred_element_type=F32)
    I1_o[...] = jnp.dot(xi, Air[...], preferred_element_type=F32)


def _run_b1(x_user, Wu, bu, Au, x_item, Wi, bi, Ai):
    bn = _NODE_BN
    full = lambda shape: pl.BlockSpec(shape, lambda r: (0, 0))
    row = lambda w: pl.BlockSpec((bn, w), lambda r: (r, 0))
    return pl.pallas_call(
        _b1_body,
        grid=(N // bn,),
        in_specs=[row(128), full((128, 16)), full((1, 16)), full((16, 64)),
                  row(128), full((128, 16)), full((1, 16)), full((16, 64))],
        out_specs=[row(16), row(16), row(64), row(64)],
        out_shape=[jax.ShapeDtypeStruct((N, 16), F32),
                   jax.ShapeDtypeStruct((N, 16), F32),
                   jax.ShapeDtypeStruct((N, 64), F32),
                   jax.ShapeDtypeStruct((N, 64), F32)],
    )(x_user, Wu, bu, Au, x_item, Wi, bi, Ai)


def _b2_body(accU, accI, xu, xi, gU, bU, gI, bI, Au2, Ai2,
             xu1_o, xi1_o, U2_o, I2_o, cu_o, ci_o):
    sU = accU[0] + accU[1]
    sI = accI[0] + accI[1]
    cntU = jnp.maximum(sU[:, 16:17], 1.0)
    cntI = jnp.maximum(sI[:, 16:17], 1.0)
    xu1 = sU[:, :16] / cntU * gU[...] + bU[...] + xu[...]
    xi1 = sI[:, :16] / cntI * gI[...] + bI[...] + xi[...]
    xu1_o[...] = xu1
    xi1_o[...] = xi1
    U2_o[...] = jnp.dot(xu1, Au2[...], preferred_element_type=F32)
    I2_o[...] = jnp.dot(xi1, Ai2[...], preferred_element_type=F32)
    cu_o[...] = cntU
    ci_o[...] = cntI


def _run_b2(accU, accI, xu, xi, gU, bU, gI, bI, Au2, Ai2):
    bn = _NODE_BN
    full = lambda shape: pl.BlockSpec(shape, lambda r: (0, 0))
    row = lambda w: pl.BlockSpec((bn, w), lambda r: (r, 0))
    acc = pl.BlockSpec((2, bn, 32), lambda r: (0, r, 0))
    return pl.pallas_call(
        _b2_body,
        grid=(N // bn,),
        in_specs=[acc, acc, row(16), row(16), full((1, 16)), full((1, 16)),
                  full((1, 16)), full((1, 16)), full((16, 64)), full((16, 64))],
        out_specs=[row(16), row(16), row(64), row(64), row(1), row(1)],
        out_shape=[jax.ShapeDtypeStruct((N, 16), F32),
                   jax.ShapeDtypeStruct((N, 16), F32),
                   jax.ShapeDtypeStruct((N, 64), F32),
                   jax.ShapeDtypeStruct((N, 64), F32),
                   jax.ShapeDtypeStruct((N, 1), F32),
                   jax.ShapeDtypeStruct((N, 1), F32)],
    )(accU, accI, xu, xi, gU, bU, gI, bI, Au2, Ai2)


def _b3_body(acc2U, acc2I, cu, ci, xu1, xi1, gU, bU, gI, bI,
             wu, wi, bens, pu_o, pi_o):
    xu2 = (acc2U[0] + acc2U[1]) / cu[...] * gU[...] + bU[...] + xu1[...]
    xi2 = (acc2I[0] + acc2I[1]) / ci[...] * gI[...] + bI[...] + xi1[...]
    pu_o[...] = jnp.dot(xu2, wu[...], preferred_element_type=F32) + bens[...]
    pi_o[...] = jnp.dot(xi2, wi[...], preferred_element_type=F32)


def _run_b3(acc2U, acc2I, cu, ci, xu1, xi1, gU, bU, gI, bI, wu, wi, bens):
    bn = _NODE_BN
    full = lambda shape: pl.BlockSpec(shape, lambda r: (0, 0))
    row = lambda w: pl.BlockSpec((bn, w), lambda r: (r, 0))
    acc = pl.BlockSpec((2, bn, 16), lambda r: (0, r, 0))
    return pl.pallas_call(
        _b3_body,
        grid=(N // bn,),
        in_specs=[acc, acc, row(1), row(1), row(16), row(16),
                  full((1, 16)), full((1, 16)), full((1, 16)), full((1, 16)),
                  full((16, 1)), full((16, 1)), full((1, 1))],
        out_specs=[row(1), row(1)],
        out_shape=[jax.ShapeDtypeStruct((N, 1), F32),
                   jax.ShapeDtypeStruct((N, 1), F32)],
    )(acc2U, acc2I, cu, ci, xu1, xi1, gU, bU, gI, bI, wu, wi, bens)


def _sigmoid(f):
    return 1.0 / (1.0 + jnp.exp(-f))


def _softplus(s):
    u = jnp.exp(-jnp.abs(s))
    p = _LP[6]
    for c in (_LP[5], _LP[4], _LP[3], _LP[2], _LP[1], _LP[0]):
        p = p * u + c
    return jnp.maximum(s, 0.0) + p


def _msg(f, s):
    return _sigmoid(f) * _softplus(s)


def _make_conv_kernel(width):
    mesh = plsc.VectorSubcoreMesh(core_axis_name="c", subcore_axis_name="s",
                                  num_cores=NC, num_subcores=NS)

    KB2 = KB // 2
    EW2 = EW // 2

    @functools.partial(
        pl.kernel,
        mesh=mesh,
        compiler_params=pltpu.CompilerParams(use_tc_tiling_on_sc=False),
        out_type=[jax.ShapeDtypeStruct((NC, NPAD, width), F32),
                  jax.ShapeDtypeStruct((NC, NPAD, width), F32)],
        scratch_types=[
            pltpu.VMEM((NBLK, KB), jnp.int32),
            pltpu.VMEM((NBLK, KB), jnp.int32),
            pltpu.VMEM((KB, 64), F32), pltpu.VMEM((KB, 64), F32),
            pltpu.VMEM((KB, 64), F32), pltpu.VMEM((KB, 64), F32),
            pltpu.VMEM((KB2, 128), F32), pltpu.VMEM((KB2, 128), F32),
            pltpu.VMEM((KB, width), F32), pltpu.VMEM((KB, width), F32),
            pltpu.VMEM((KB, width), F32), pltpu.VMEM((KB, width), F32),
            pltpu.VMEM((NPS, width), F32),
            pltpu.VMEM_SHARED((NPAD, width), F32),
            pltpu.VMEM_SHARED((NPAD, width), F32),
            pltpu.SemaphoreType.DMA, pltpu.SemaphoreType.DMA,
            pltpu.SemaphoreType.DMA, pltpu.SemaphoreType.DMA,
            pltpu.SemaphoreType.DMA, pltpu.SemaphoreType.DMA,
            pltpu.SemaphoreType.DMA, pltpu.SemaphoreType.DMA,
            pltpu.SemaphoreType.DMA, pltpu.SemaphoreType.DMA,
        ],
    )
    def conv(uidx_hbm, iidx_hbm, t_hbm, U_hbm, I_hbm,
             accU_hbm, accI_hbm,
             idxU, idxI, rU0, rU1, rI0, rI1, tb0, tb1,
             mU0, mU1, mI0, mI1, zbuf,
             sAccU, sAccI, sU0, sU1, sI0, sI1, sT0, sT1,
             sSU0, sSU1, sSI0, sSI1):
        cid = lax.axis_index("c")
        sid = lax.axis_index("s")
        wid = sid * NC + cid

        zero16 = jnp.zeros((16,), F32)

        @plsc.parallel_loop(0, NPS, unroll=8)
        def zrow(r):
            for j in range(width // 16):
                zbuf[r, pl.ds(j * 16, 16)] = zero16

        zb = sid * NPS
        pltpu.sync_copy(zbuf, sAccU.at[pl.ds(zb, NPS)])
        pltpu.sync_copy(zbuf, sAccI.at[pl.ds(zb, NPS)])

        pltpu.sync_copy(uidx_hbm.at[wid], idxU)
        pltpu.sync_copy(iidx_hbm.at[wid], idxI)

        if width == 32:
            lanes = lax.iota(jnp.int32, 16)
            onehot = jnp.where(lanes == 0, 1.0, 0.0).astype(F32)

            @plsc.parallel_loop(0, KB, unroll=8)
            def initrow(k):
                for m in (mU0, mU1, mI0, mI1):
                    m[k, pl.ds(16, 16)] = onehot

        plsc.subcore_barrier()

        pbase = wid * EW2

        def gathers(b, rU, rI, tb, sU, sI, sT):
            gu = pltpu.make_async_copy(U_hbm.at[idxU.at[b]], rU, sU)
            gi = pltpu.make_async_copy(I_hbm.at[idxI.at[b]], rI, sI)
            gt = pltpu.make_async_copy(t_hbm.at[pl.ds(pbase + b * KB2, KB2)],
                                       tb, sT)
            return gu, gi, gt

        def issue(b, rU, rI, tb, sU, sI, sT):
            for c in gathers(b, rU, rI, tb, sU, sI, sT):
                c.start()

        def wait(b, rU, rI, tb, sU, sI, sT):
            for c in gathers(b, rU, rI, tb, sU, sI, sT):
                c.wait()

        def scat(b, mU, mI, sSU, sSI):
            cu = pltpu.make_async_copy(mU, sAccU.at[idxU.at[b]], sSU)
            ci = pltpu.make_async_copy(mI, sAccI.at[idxI.at[b]], sSI)
            return cu, ci

        def compute_scatter(b, rU, rI, tb, mU, mI, sSU, sSI):
            msgU, msgI = mU, mI

            @plsc.parallel_loop(0, KB2, unroll=4)
            def pair(p):
                for h in range(2):
                    k = p * 2 + h
                    tfu = tb[p, pl.ds(64 * h + 0, 16)]
                    tsu = tb[p, pl.ds(64 * h + 16, 16)]
                    tfi = tb[p, pl.ds(64 * h + 32, 16)]
                    tsi = tb[p, pl.ds(64 * h + 48, 16)]
                    ufd = rU[k, pl.ds(0, 16)]
                    usd = rU[k, pl.ds(16, 16)]
                    ufs = rU[k, pl.ds(32, 16)]
                    uss = rU[k, pl.ds(48, 16)]
                    ifd = rI[k, pl.ds(0, 16)]
                    isd = rI[k, pl.ds(16, 16)]
                    ifs = rI[k, pl.ds(32, 16)]
                    iss = rI[k, pl.ds(48, 16)]
                    msgI[k, pl.ds(0, 16)] = _msg(tfu + ifd + ufs,
                                                 tsu + isd + uss)
                    msgU[k, pl.ds(0, 16)] = _msg(tfi + ufd + ifs,
                                                 tsi + usd + iss)
            pltpu.async_copy(msgU, sAccU.at[idxU.at[b]], sSU, add=True)
            pltpu.async_copy(msgI, sAccI.at[idxI.at[b]], sSI, add=True)

        issue(0, rU0, rI0, tb0, sU0, sI0, sT0)

        def pipeline(g, c):
            b0 = 2 * g
            b1 = b0 + 1
            b2 = b0 + 2
            wait(b0, rU0, rI0, tb0, sU0, sI0, sT0)
            issue(b1, rU1, rI1, tb1, sU1, sI1, sT1)

            @pl.when(g > 0)
            def _():
                for c_ in scat(b0 - 2, mU0, mI0, sSU0, sSI0):
                    c_.wait()
            compute_scatter(b0, rU0, rI0, tb0, mU0, mI0, sSU0, sSI0)
            wait(b1, rU1, rI1, tb1, sU1, sI1, sT1)
            issue(b2, rU0, rI0, tb0, sU0, sI0, sT0)

            @pl.when(g > 0)
            def _():
                for c_ in scat(b1 - 2, mU1, mI1, sSU1, sSI1):
                    c_.wait()
            compute_scatter(b1, rU1, rI1, tb1, mU1, mI1, sSU1, sSI1)
            return c
        lax.fori_loop(0, NBLK // 2, pipeline, 0)
        wait(NBLK - 1, rU0, rI0, tb0, sU0, sI0, sT0)
        for c_ in scat(NBLK - 3, mU0, mI0, sSU0, sSI0):
            c_.wait()
        compute_scatter(NBLK - 1, rU0, rI0, tb0, mU0, mI0, sSU0, sSI0)
        for c_ in scat(NBLK - 2, mU1, mI1, sSU1, sSI1):
            c_.wait()
        for c_ in scat(NBLK - 1, mU0, mI0, sSU0, sSI0):
            c_.wait()

        plsc.subcore_barrier()

        pltpu.sync_copy(sAccU.at[pl.ds(zb, NPS)],
                        accU_hbm.at[cid, pl.ds(zb, NPS)])
        pltpu.sync_copy(sAccI.at[pl.ds(zb, NPS)],
                        accI_hbm.at[cid, pl.ds(zb, NPS)])

    return conv


_make_conv_kernel = functools.lru_cache(maxsize=None)(_make_conv_kernel)


def _make_ens_kernel():
    mesh = plsc.VectorSubcoreMesh(core_axis_name="c", subcore_axis_name="s",
                                  num_cores=NC, num_subcores=NS)

    @functools.partial(
        pl.kernel,
        mesh=mesh,
        compiler_params=pltpu.CompilerParams(needs_layout_passes=False),
        out_type=jax.ShapeDtypeStruct((BATCH,), F32),
        scratch_types=[
            pltpu.VMEM((N,), F32), pltpu.VMEM((N,), F32),
            pltpu.VMEM((BW,), jnp.int32), pltpu.VMEM((BW,), jnp.int32),
            pltpu.VMEM((BW,), F32),
        ],
    )
    def ens(pu_hbm, pi_hbm, ebu_hbm, ebi_hbm, out_hbm,
            puv, piv, idxu, idxi, outv):
        cid = lax.axis_index("c")
        sid = lax.axis_index("s")
        wid = sid * NC + cid
        base = wid * BW
        pltpu.sync_copy(pu_hbm, puv)
        pltpu.sync_copy(pi_hbm, piv)
        pltpu.sync_copy(ebu_hbm.at[pl.ds(base, BW)], idxu)
        pltpu.sync_copy(ebi_hbm.at[pl.ds(base, BW)], idxi)

        def chunk(j, c):
            iu = idxu[pl.ds(j * 16, 16)]
            ii = idxi[pl.ds(j * 16, 16)]
            a = plsc.load_gather(puv, [iu])
            b = plsc.load_gather(piv, [ii])
            outv[pl.ds(j * 16, 16)] = _sigmoid(a + b)
            return c
        lax.fori_loop(0, BW // 16, chunk, 0)
        pltpu.sync_copy(outv, out_hbm.at[pl.ds(base, BW)])

    return ens


_make_ens_kernel = functools.lru_cache(maxsize=None)(_make_ens_kernel)


def kernel(x_user, x_item, edge_attr, edge_index, edge_index_batch,
           We1, be1, We2, be2, We3, be3, Wu, bu, Wi, bi,
           c1ui_Wf, c1ui_bf, c1ui_Ws, c1ui_bs, c1ui_g, c1ui_b,
           c1iu_Wf, c1iu_bf, c1iu_Ws, c1iu_bs, c1iu_g, c1iu_b,
           c2ui_Wf, c2ui_bf, c2ui_Ws, c2ui_bs, c2ui_g, c2ui_b,
           c2iu_Wf, c2iu_bf, c2iu_Ws, c2iu_bs, c2iu_g, c2iu_b,
           Wens, bens):
    def epart(Wf, Ws):
        return jnp.concatenate([Wf[2 * H:], Ws[2 * H:]], axis=1)

    S1 = jnp.concatenate([epart(c1ui_Wf, c1ui_Ws),
                          epart(c1iu_Wf, c1iu_Ws)], axis=1)
    S2 = jnp.concatenate([epart(c2ui_Wf, c2ui_Ws),
                          epart(c2iu_Wf, c2iu_Ws)], axis=1)
    Wt1 = We3 @ S1
    Wt2 = We3 @ S2
    bt1 = (be3 @ S1 + jnp.concatenate([c1ui_bf, c1ui_bs, c1iu_bf, c1iu_bs])
           )[None, :]
    bt2 = (be3 @ S2 + jnp.concatenate([c2ui_bf, c2ui_bs, c2iu_bf, c2iu_bs])
           )[None, :]

    def tables(Wf_ud, Ws_ud, Wf_du, Ws_du):
        Au = jnp.concatenate([Wf_du[:H], Ws_du[:H],
                              Wf_ud[H:2 * H], Ws_ud[H:2 * H]], axis=1)
        Ai = jnp.concatenate([Wf_ud[:H], Ws_ud[:H],
                              Wf_du[H:2 * H], Ws_du[H:2 * H]], axis=1)
        return Au, Ai

    Au1, Ai1 = tables(c1ui_Wf, c1ui_Ws, c1iu_Wf, c1iu_Ws)
    Au2, Ai2 = tables(c2ui_Wf, c2ui_Ws, c2iu_Wf, c2iu_Ws)

    inv = 1.0 / jnp.sqrt(jnp.float32(1.0 + 1e-5))
    g1U, b1U = (c1iu_g * inv)[None, :], c1iu_b[None, :]
    g1I, b1I = (c1ui_g * inv)[None, :], c1ui_b[None, :]
    g2U, b2U = (c2iu_g * inv)[None, :], c2iu_b[None, :]
    g2I, b2I = (c2ui_g * inv)[None, :], c2ui_b[None, :]

    uidx = edge_index[0].astype(jnp.int32).reshape(NW, NBLK, KB)
    iidx = edge_index[1].astype(jnp.int32).reshape(NW, NBLK, KB)
    ebu = edge_index_batch[:, 0].astype(jnp.int32)
    ebi = edge_index_batch[:, 1].astype(jnp.int32)

    eye2 = jnp.eye(2, dtype=F32)
    We1p = jnp.kron(eye2, We1)
    We2p = jnp.kron(eye2, We2)
    Wt1p = jnp.kron(eye2, Wt1)
    Wt2p = jnp.kron(eye2, Wt2)
    be1p = jnp.tile(be1, 2)[None, :]
    be2p = jnp.tile(be2, 2)[None, :]
    bt1p = jnp.tile(bt1[0], 2)[None, :]
    bt2p = jnp.tile(bt2[0], 2)[None, :]
    ea2 = edge_attr.reshape(E // 2, 32)

    t1 = _run_mlp(ea2, We1p, be1p, We2p, be2p, Wt1p, bt1p)
    xu, xi, U1, I1 = _run_b1(x_user, Wu, bu[None, :], Au1,
                             x_item, Wi, bi[None, :], Ai1)

    accU, accI = _make_conv_kernel(32)(uidx, iidx, t1, U1, I1)

    t2 = _run_mlp(ea2, We1p, be1p, We2p, be2p, Wt2p, bt2p)

    xu1, xi1, U2, I2, cu, ci = _run_b2(accU[:, :N], accI[:, :N], xu, xi,
                                       g1U, b1U, g1I, b1I, Au2, Ai2)

    acc2U, acc2I = _make_conv_kernel(16)(uidx, iidx, t2, U2, I2)

    pu, pi = _run_b3(acc2U[:, :N], acc2I[:, :N], cu, ci, xu1, xi1,
                     g2U, b2U, g2I, b2I,
                     Wens[:H], Wens[H:], bens[None, :])

    out = _make_ens_kernel()(pu.reshape(N), pi.reshape(N), ebu, ebi)
    return out.reshape(BATCH, 1)
